# baseline (speedup 1.0000x reference)
import functools

import jax
import jax.numpy as jnp
from jax import lax
from jax.experimental import pallas as pl
from jax.experimental.pallas import tpu as pltpu
from jax.experimental.pallas import tpu_sc as plsc

_NC = 2
_NS = 16
_L = 16
_NW = _NC * _NS

_H0 = 104
_H1 = 96
_VP2 = 489 * 1024


def _make_pool_kernel(B, D, H, denom):
    b_per_w = B // _NW
    CH = 64
    n_ch = b_per_w // CH
    NB = 8
    n_grp = 2 * CH // NB
    UN = 8
    scale = jnp.float32(1.0 / denom)
    mesh = plsc.VectorSubcoreMesh(
        core_axis_name="c", subcore_axis_name="s",
        num_cores=_NC, num_subcores=_NS)

    def half_idx(idx_v, r, p):
        if p == 0:
            return idx_v.at[r, pl.ds(0, _H0)]
        return idx_v.at[r, pl.ds(_H0, _H1)]

    def half_buf(buf, p):
        return buf.at[pl.ds(0, _H0)] if p == 0 else buf.at[pl.ds(0, _H1)]

    @functools.partial(
        pl.kernel,
        mesh=mesh,
        compiler_params=pltpu.CompilerParams(
            use_tc_tiling_on_sc=False, needs_layout_passes=False),
        out_type=jax.ShapeDtypeStruct((B, D), jnp.float32),
        scratch_types=[
            pltpu.VMEM((CH, H), jnp.int32),
            pltpu.VMEM((CH, H + 8), jnp.int32),
            [pltpu.VMEM((_H0, 2 * D), jnp.bfloat16) for _ in range(NB)],
            pltpu.VMEM((CH, D), jnp.float32),
            [pltpu.SemaphoreType.DMA for _ in range(NB)],
        ],
    )
    def pool_k(idx2_hbm, off_hbm, emb2_hbm, out_hbm,
               idx_v, off_v, bufs, pooled_v, sems):
        wid = lax.axis_index("s") * _NC + lax.axis_index("c")
        base = wid * b_per_w
        ev = lax.iota(jnp.int32, _L) * 2

        def accumulate(buf, off_row, j_base, nrows, init):
            def body(jj, accs):
                j0 = jj * UN
                o_vec = off_row[pl.ds(j_base + j0, _L)]
                new = list(accs)
                for u in range(UN):
                    o = o_vec[u]
                    for g in range(2):
                        v = buf[j0 + u, pl.ds(o + 32 * g, 32)]
                        a, b = plsc.unpack(
                            v, format=plsc.PackFormat.INTERLEAVED)
                        c = 4 * (u & 1) + 2 * g
                        new[c] = new[c] + a
                        new[c + 1] = new[c + 1] + b
                return tuple(new)
            return lax.fori_loop(0, nrows // UN, body, init)

        zeros = tuple(jnp.zeros((_L,), jnp.float32) for _ in range(8))

        def chunk_body(c, carry):
            row0 = base + c * CH
            pltpu.sync_copy(idx2_hbm.at[pl.ds(row0, CH)], idx_v)
            pltpu.sync_copy(off_hbm.at[pl.ds(row0, CH)], off_v)
            for b in range(NB):
                pltpu.async_copy(
                    emb2_hbm.at[half_idx(idx_v, b // 2, b & 1)],
                    half_buf(bufs[b], b & 1), sems[b])

            def grp_body(g, carry2):
                for b in range(NB):
                    r = g * (NB // 2) + (b // 2)
                    p = b & 1
                    pltpu.make_async_copy(
                        emb2_hbm.at[half_idx(idx_v, r, p)],
                        half_buf(bufs[b], p), sems[b]).wait()
                    off_row = off_v.at[r]
                    if p == 0:
                        pair_accs = accumulate(bufs[b], off_row, 0, _H0, zeros)
                    else:
                        accs = accumulate(bufs[b], off_row, _H0, _H1, pair_accs)
                        row_ref = pooled_v.at[r]
                        for gg in range(2):
                            a_sum = (accs[2 * gg] + accs[4 + 2 * gg]) * scale
                            b_sum = (accs[2 * gg + 1]
                                     + accs[4 + 2 * gg + 1]) * scale
                            plsc.store_scatter(row_ref, [ev + 32 * gg], a_sum)
                            plsc.store_scatter(
                                row_ref, [ev + 32 * gg + 1], b_sum)

                    @pl.when(g < n_grp - 1)
                    def _issue():
                        pltpu.async_copy(
                            emb2_hbm.at[half_idx(idx_v, r + NB // 2, p)],
                            half_buf(bufs[b], p), sems[b])
                return carry2

            lax.fori_loop(0, n_grp, grp_body, 0)
            pltpu.sync_copy(pooled_v, out_hbm.at[pl.ds(row0, CH)])
            return carry

        lax.fori_loop(0, n_ch, chunk_body, 0)

    return pool_k


def _pack_block_kernel(a_ref, b_ref, o_ref):
    ta = jnp.transpose(a_ref[...]).astype(jnp.bfloat16)
    tb = jnp.transpose(b_ref[...]).astype(jnp.bfloat16)
    o_ref[...] = jnp.concatenate([ta, tb], axis=1)


def _pack_table(emb_t, D):
    V = emb_t.shape[1]
    blk = 1024
    n_blk = _VP2 // blk
    hi_max = V // blk
    return pl.pallas_call(
        _pack_block_kernel,
        grid=(n_blk,),
        in_specs=[
            pl.BlockSpec((D, blk), lambda i: (0, i)),
            pl.BlockSpec((D, blk), lambda i: (0, jnp.minimum(i + n_blk, hi_max))),
        ],
        out_specs=pl.BlockSpec((blk, 2 * D), lambda i: (i, 0)),
        out_shape=jax.ShapeDtypeStruct((_VP2, 2 * D), jnp.bfloat16),
    )(emb_t, emb_t)


def _mlp_block_kernel(p_ref, w1t_ref, b1_ref, w2t_ref, b2_ref, o_ref):
    h = jnp.dot(p_ref[...], w1t_ref[...], preferred_element_type=jnp.float32)
    h = jnp.maximum(h + b1_ref[...], 0.0)
    o = jnp.dot(h, w2t_ref[...], preferred_element_type=jnp.float32)
    o_ref[...] = o + b2_ref[...]


def _mlp(pooled, W1t, b1, W2t, b2, block_b=2048):
    B, D = pooled.shape
    N = W2t.shape[1]
    grid = (B // block_b,)
    return pl.pallas_call(
        _mlp_block_kernel,
        grid=grid,
        in_specs=[
            pl.BlockSpec((block_b, D), lambda i: (i, 0)),
            pl.BlockSpec((D, D), lambda i: (0, 0)),
            pl.BlockSpec((1, D), lambda i: (0, 0)),
            pl.BlockSpec((D, N), lambda i: (0, 0)),
            pl.BlockSpec((1, N), lambda i: (0, 0)),
        ],
        out_specs=pl.BlockSpec((block_b, N), lambda i: (i, 0)),
        out_shape=jax.ShapeDtypeStruct((B, N), jnp.float32),
    )(pooled, W1t, b1, W2t, b2)


def kernel(x, seq_lens, emb, W1, b1, W2, b2):
    del seq_lens
    B, H = x.shape
    V, D = emb.shape
    x = x.astype(jnp.int32)
    big = x >= _VP2
    idx2 = jnp.where(big, x - _VP2, x)
    off = jnp.concatenate(
        [jnp.where(big, D, 0).astype(jnp.int32),
         jnp.zeros((B, 8), jnp.int32)], axis=1)
    emb2 = _pack_table(emb.T, D)
    pooled = _make_pool_kernel(B, D, H, float(H))(idx2, off, emb2)
    return _mlp(pooled, W1.T, b1.reshape(1, D), W2.T, b2.reshape(1, -1))


# speedup vs baseline: 1.3143x; 1.3143x over previous
import functools

import jax
import jax.numpy as jnp
from jax import lax
from jax.experimental import pallas as pl
from jax.experimental.pallas import tpu as pltpu
from jax.experimental.pallas import tpu_sc as plsc

_NC = 2
_NS = 16
_L = 16
_NW = _NC * _NS

_H0 = 104
_H1 = 96
_BLK = 512
_NBLK = 489
_Q = _NBLK * _BLK


def _pack_block_kernel(a_ref, b_ref, c_ref, d_ref, o_ref):
    parts = []
    for ref in (a_ref, b_ref, c_ref, d_ref):
        u = jax.lax.bitcast_convert_type(ref[...], jnp.uint32)
        b16 = (u + 0x7FFF + ((u >> 16) & 1)) >> 16
        u3 = b16.reshape(32, 2, _BLK)
        lo = u3[:, 0, :]
        hi = u3[:, 1, :]
        packed = lo | (hi << 16)
        parts.append(jnp.transpose(packed).astype(jnp.int32))
    o_ref[...] = jnp.concatenate(parts, axis=1)


def _pack_table(emb_t, D):
    V = emb_t.shape[1]
    hi_max = (V + _BLK - 1) // _BLK - 1
    specs = []
    for k in range(4):
        specs.append(pl.BlockSpec(
            (D, _BLK),
            functools.partial(
                lambda i, kk: (0, jnp.minimum(i + kk * _NBLK, hi_max)), kk=k)))
    return pl.pallas_call(
        _pack_block_kernel,
        grid=(_NBLK,),
        in_specs=specs,
        out_specs=pl.BlockSpec((_BLK, 4 * 32), lambda i: (i, 0)),
        out_shape=jax.ShapeDtypeStruct((_Q, 4 * 32), jnp.int32),
    )(emb_t, emb_t, emb_t, emb_t)


def _make_pool_kernel(B, D, H, denom):
    b_per_w = B // _NW
    CH = 64
    n_ch = b_per_w // CH
    NB = 8
    n_grp = 2 * CH // NB
    UN = 8
    scale = jnp.float32(1.0 / denom)
    mesh = plsc.VectorSubcoreMesh(
        core_axis_name="c", subcore_axis_name="s",
        num_cores=_NC, num_subcores=_NS)

    def half_idx(idx_v, r, p):
        if p == 0:
            return idx_v.at[r, pl.ds(0, _H0)]
        return idx_v.at[r, pl.ds(_H0, _H1)]

    def half_buf(buf, p):
        return buf.at[pl.ds(0, _H0)] if p == 0 else buf.at[pl.ds(0, _H1)]

    @functools.partial(
        pl.kernel,
        mesh=mesh,
        compiler_params=pltpu.CompilerParams(
            use_tc_tiling_on_sc=False, needs_layout_passes=False),
        out_type=jax.ShapeDtypeStruct((B, D), jnp.float32),
        scratch_types=[
            pltpu.VMEM((CH, H), jnp.int32),
            pltpu.VMEM((CH, H + 8), jnp.int32),
            [pltpu.VMEM((_H0, D), jnp.int32) for _ in range(NB)],
            pltpu.VMEM((CH, D), jnp.float32),
            [pltpu.SemaphoreType.DMA for _ in range(NB)],
        ],
    )
    def pool_k(idx2_hbm, off_hbm, emb2_hbm, out_hbm,
               idx_v, off_v, bufs, pooled_v, sems):
        wid = lax.axis_index("s") * _NC + lax.axis_index("c")
        base = wid * b_per_w
        ev = lax.iota(jnp.int32, _L) * 2

        def accumulate(buf, off_row, j_base, nrows, init):
            def body(jj, accs):
                j0 = jj * UN
                o_vec = off_row[pl.ds(j_base + j0, _L)]
                new = list(accs)
                for u in range(UN):
                    o = o_vec[u]
                    for g in range(2):
                        w = buf[j0 + u, pl.ds(o + _L * g, _L)]
                        v = plsc.bitcast(w, jnp.bfloat16)
                        a, b = plsc.unpack(
                            v, format=plsc.PackFormat.INTERLEAVED)
                        c = 4 * (u & 1) + 2 * g
                        new[c] = new[c] + a
                        new[c + 1] = new[c + 1] + b
                return tuple(new)
            return lax.fori_loop(0, nrows // UN, body, init)

        zeros = tuple(jnp.zeros((_L,), jnp.float32) for _ in range(8))

        def chunk_body(c, carry):
            row0 = base + c * CH
            pltpu.sync_copy(idx2_hbm.at[pl.ds(row0, CH)], idx_v)
            pltpu.sync_copy(off_hbm.at[pl.ds(row0, CH)], off_v)
            for b in range(NB):
                pltpu.async_copy(
                    emb2_hbm.at[half_idx(idx_v, b // 2, b & 1)],
                    half_buf(bufs[b], b & 1), sems[b])

            def grp_body(g, carry2):
                for b in range(NB):
                    r = g * (NB // 2) + (b // 2)
                    p = b & 1
                    pltpu.make_async_copy(
                        emb2_hbm.at[half_idx(idx_v, r, p)],
                        half_buf(bufs[b], p), sems[b]).wait()
                    off_row = off_v.at[r]
                    if p == 0:
                        pair_accs = accumulate(bufs[b], off_row, 0, _H0, zeros)
                    else:
                        accs = accumulate(bufs[b], off_row, _H0, _H1, pair_accs)
                        row_ref = pooled_v.at[r]
                        for gg in range(2):
                            a_sum = (accs[2 * gg] + accs[4 + 2 * gg]) * scale
                            b_sum = (accs[2 * gg + 1]
                                     + accs[4 + 2 * gg + 1]) * scale
                            plsc.store_scatter(row_ref, [ev + 32 * gg], a_sum)
                            plsc.store_scatter(
                                row_ref, [ev + 32 * gg + 1], b_sum)

                    @pl.when(g < n_grp - 1)
                    def _issue():
                        pltpu.async_copy(
                            emb2_hbm.at[half_idx(idx_v, r + NB // 2, p)],
                            half_buf(bufs[b], p), sems[b])
                return carry2

            lax.fori_loop(0, n_grp, grp_body, 0)
            pltpu.sync_copy(pooled_v, out_hbm.at[pl.ds(row0, CH)])
            return carry

        lax.fori_loop(0, n_ch, chunk_body, 0)

    return pool_k


def _mlp_block_kernel(p_ref, w1t_ref, b1_ref, w2t_ref, b2_ref, o_ref):
    h = jnp.dot(p_ref[...], w1t_ref[...], preferred_element_type=jnp.float32)
    h = jnp.maximum(h + b1_ref[...], 0.0)
    o = jnp.dot(h, w2t_ref[...], preferred_element_type=jnp.float32)
    o_ref[...] = o + b2_ref[...]


def _mlp(pooled, W1t, b1, W2t, b2, block_b=2048):
    B, D = pooled.shape
    N = W2t.shape[1]
    grid = (B // block_b,)
    return pl.pallas_call(
        _mlp_block_kernel,
        grid=grid,
        in_specs=[
            pl.BlockSpec((block_b, D), lambda i: (i, 0)),
            pl.BlockSpec((D, D), lambda i: (0, 0)),
            pl.BlockSpec((1, D), lambda i: (0, 0)),
            pl.BlockSpec((D, N), lambda i: (0, 0)),
            pl.BlockSpec((1, N), lambda i: (0, 0)),
        ],
        out_specs=pl.BlockSpec((block_b, N), lambda i: (i, 0)),
        out_shape=jax.ShapeDtypeStruct((B, N), jnp.float32),
    )(pooled, W1t, b1, W2t, b2)


def kernel(x, seq_lens, emb, W1, b1, W2, b2):
    del seq_lens
    B, H = x.shape
    V, D = emb.shape
    x = x.astype(jnp.int32)
    s = 4 * (x % _Q) + x // _Q
    idx2 = s >> 1
    off = jnp.concatenate(
        [(s & 1) << 5, jnp.zeros((B, 8), jnp.int32)], axis=1)
    packed = _pack_table(emb.T, D)
    emb2 = packed.reshape(2 * _Q, D)
    pooled = _make_pool_kernel(B, D, H, float(H))(idx2, off, emb2)
    return _mlp(pooled, W1.T, b1.reshape(1, D), W2.T, b2.reshape(1, -1))
